# baseline (speedup 1.0000x reference)
import jax
import jax.numpy as jnp
from jax.experimental import pallas as pl
from jax.experimental.pallas import tpu as pltpu

_ROWS = 128
_N = 32768
_BLK = 32768
_SUB = 256


def _body(x_ref, t_ref, o_ref, carry_ref):
    i = pl.program_id(0)

    @pl.when(i == 0)
    def _init():
        carry_ref[...] = jnp.zeros_like(carry_ref)

    carry = carry_ref[...]
    for k in range(_BLK // _SUB):
        x = x_ref[:, k * _SUB:(k + 1) * _SUB]
        pre = jax.lax.dot(
            x.astype(jnp.bfloat16), t_ref[...],
            preferred_element_type=jnp.float32)
        o_ref[:, k * _SUB:(k + 1) * _SUB] = pre + carry
        carry = carry + jnp.sum(x, axis=1, keepdims=True)
    carry_ref[...] = carry


def kernel(x):
    rows, n = x.shape
    grid = (n // _BLK,)
    tri = jnp.triu(jnp.ones((_SUB, _SUB), dtype=jnp.bfloat16))
    return pl.pallas_call(
        _body,
        grid=grid,
        in_specs=[
            pl.BlockSpec((rows, _BLK), lambda i: (0, i)),
            pl.BlockSpec((_SUB, _SUB), lambda i: (0, 0)),
        ],
        out_specs=pl.BlockSpec((rows, _BLK), lambda i: (0, i)),
        out_shape=jax.ShapeDtypeStruct((rows, n), jnp.float32),
        scratch_shapes=[pltpu.VMEM((rows, 1), jnp.float32)],
    )(x, tri)


# speedup vs baseline: 1.2790x; 1.2790x over previous
import jax
import jax.numpy as jnp
from jax.experimental import pallas as pl
from jax.experimental.pallas import tpu as pltpu

_ROWS = 128
_N = 32768
_CH = 4096
_NCHUNK = _N // _CH
_NBUF = 3
_SUB = 256


def _body(x_hbm, t_ref, o_hbm, *scratch):
    inbufs = scratch[:_NBUF]
    outbufs = scratch[_NBUF:2 * _NBUF]
    insems = scratch[2 * _NBUF:2 * _NBUF + _NBUF]
    outsems = scratch[2 * _NBUF + _NBUF:]
    tri = t_ref[...]

    def in_copy(c):
        return pltpu.make_async_copy(
            x_hbm.at[:, pl.ds(c * _CH, _CH)], inbufs[c % _NBUF],
            insems[c % _NBUF])

    def out_copy(c):
        return pltpu.make_async_copy(
            outbufs[c % _NBUF], o_hbm.at[:, pl.ds(c * _CH, _CH)],
            outsems[c % _NBUF])

    for c in range(_NBUF):
        in_copy(c).start()

    carry = jnp.zeros((_ROWS, 1), jnp.float32)
    for c in range(_NCHUNK):
        in_copy(c).wait()
        if c >= _NBUF:
            out_copy(c - _NBUF).wait()
        xb = inbufs[c % _NBUF]
        ob = outbufs[c % _NBUF]
        for k in range(_CH // _SUB):
            x = xb[:, k * _SUB:(k + 1) * _SUB]
            pre = jax.lax.dot(
                x.astype(jnp.bfloat16), tri,
                preferred_element_type=jnp.float32)
            ob[:, k * _SUB:(k + 1) * _SUB] = pre + carry
            carry = carry + jnp.sum(x, axis=1, keepdims=True)
        out_copy(c).start()
        if c + _NBUF < _NCHUNK:
            in_copy(c + _NBUF).start()
    for c in range(max(_NCHUNK - _NBUF, 0), _NCHUNK):
        out_copy(c).wait()


def kernel(x):
    rows, n = x.shape
    tri = jnp.triu(jnp.ones((_SUB, _SUB), dtype=jnp.bfloat16))
    return pl.pallas_call(
        _body,
        in_specs=[
            pl.BlockSpec(memory_space=pltpu.MemorySpace.HBM),
            pl.BlockSpec((_SUB, _SUB), lambda: (0, 0)),
        ],
        out_specs=pl.BlockSpec(memory_space=pltpu.MemorySpace.HBM),
        out_shape=jax.ShapeDtypeStruct((rows, n), jnp.float32),
        scratch_shapes=(
            [pltpu.VMEM((_ROWS, _CH), jnp.float32) for _ in range(_NBUF)]
            + [pltpu.VMEM((_ROWS, _CH), jnp.float32) for _ in range(_NBUF)]
            + [pltpu.SemaphoreType.DMA for _ in range(2 * _NBUF)]
        ),
    )(x, tri)
